# baseline (speedup 1.0000x reference)
import jax
import jax.numpy as jnp
from jax.experimental import pallas as pl
from jax.experimental.pallas import tpu as pltpu

IN_N = 4096
OUT_N = 1024
DELAYS = 16
V_TH = 1.0
TAU = 5.0

COLS = IN_N * DELAYS
BI = 4096
NSTEP = COLS // BI


def _body(t_ref, xe_ref, de_ref, w_ref, s_ref, out_ref, acc_ref):
    c = pl.program_id(0)
    tval = t_ref[0, 0]
    xx = xe_ref[0]
    tt = tval - xx - de_ref[...]
    mask = jnp.logical_and(xx != -1.0, tt >= 0.0)
    o_row = jnp.where(mask, tt * jnp.exp(1.0 - tt / TAU) / TAU, 0.0)

    part = jnp.sum(w_ref[...] * o_row, axis=1, keepdims=True)

    @pl.when(c == 0)
    def _():
        acc_ref[...] = jnp.zeros_like(acc_ref)

    acc_ref[...] += part

    @pl.when(c == NSTEP - 1)
    def _():
        v = acc_ref[...]
        s_old = s_ref[...]
        fire = jnp.logical_and(s_old < 0.0, v >= V_TH)
        out_ref[...] = jnp.where(fire, tval, s_old)


def kernel(t, x, w, d, s):
    w2 = w.reshape(OUT_N, COLS)
    xe = jnp.repeat(x, DELAYS).reshape(NSTEP, 1, BI)
    de = jnp.tile(d, BI // DELAYS).reshape(1, BI)
    t2 = jnp.asarray(t, jnp.float32).reshape(1, 1)
    s2 = s.reshape(OUT_N, 1)

    out = pl.pallas_call(
        _body,
        grid=(NSTEP,),
        in_specs=[
            pl.BlockSpec((1, 1), lambda c: (0, 0)),
            pl.BlockSpec((1, 1, BI), lambda c: (c, 0, 0)),
            pl.BlockSpec((1, BI), lambda c: (0, 0)),
            pl.BlockSpec((OUT_N, BI), lambda c: (0, c)),
            pl.BlockSpec((OUT_N, 1), lambda c: (0, 0)),
        ],
        out_specs=pl.BlockSpec((OUT_N, 1), lambda c: (0, 0)),
        out_shape=jax.ShapeDtypeStruct((OUT_N, 1), jnp.float32),
        scratch_shapes=[pltpu.VMEM((OUT_N, 1), jnp.float32)],
    )(t2, xe, de, w2, s2)
    return out.reshape(OUT_N)
